# Initial kernel scaffold; baseline (speedup 1.0000x reference)
import jax
import jax.numpy as jnp
from jax.experimental import pallas as pl


def kernel(fmap1, fmap2, coords, sigma):
    raise NotImplementedError("write your pallas kernel here")


# speedup vs baseline: 1.6542x; 1.6542x over previous
import functools
import math

import jax
import jax.numpy as jnp
from jax import lax
from jax.experimental import pallas as pl
from jax.experimental.pallas import tpu as pltpu
from jax.experimental.pallas import tpu_sc as plsc

_SAMPLES = 9
_NLVL = 4
_G = 4
_GS = _G * _SAMPLES
_CH = _NLVL * _GS
_LANES = 16


def _corr_pyr_body(f1_ref, f2_ref, o0_ref, o1_ref, o2_ref, o3_ref):
    f1 = f1_ref[0, 0]
    f2 = f2_ref[0, 0]
    c = f1.shape[0]
    scale = jnp.float32(1.0 / math.sqrt(float(c)))
    corr = lax.dot_general(f1, f2, (((0,), (0,)), ((), ())),
                           preferred_element_type=jnp.float32) * scale
    outs = [o0_ref, o1_ref, o2_ref, o3_ref]
    cur = corr
    outs[0][0, 0] = cur
    for lvl in range(1, _NLVL):
        wl = cur.shape[1]
        wn = wl // 2
        r = lax.broadcasted_iota(jnp.int32, (wl, wn), 0)
        cc = lax.broadcasted_iota(jnp.int32, (wl, wn), 1)
        pool = jnp.where(r // 2 == cc, jnp.float32(0.5), jnp.float32(0.0))
        cur = lax.dot_general(cur, pool, (((1,), (0,)), ((), ())),
                              preferred_element_type=jnp.float32)
        outs[lvl][0, 0] = cur


def _corr_pyramid(f1t, f2t):
    b, h, c, w = f1t.shape
    level_w = [w // (2 ** l) for l in range(_NLVL)]
    return pl.pallas_call(
        _corr_pyr_body,
        grid=(b, h),
        in_specs=[pl.BlockSpec((1, 1, c, w), lambda i, j: (i, j, 0, 0))] * 2,
        out_specs=[pl.BlockSpec((1, 1, w, wl), lambda i, j: (i, j, 0, 0))
                   for wl in level_w],
        out_shape=[jax.ShapeDtypeStruct((b, h, w, wl), jnp.float32)
                   for wl in level_w],
    )(f1t, f2t)


def _make_sc_sampler(n_chunks, level_w):
    mesh = plsc.VectorSubcoreMesh(core_axis_name="c", subcore_axis_name="s")
    n_workers = 32
    chunks_per_w = n_chunks // n_workers

    @functools.partial(
        pl.kernel,
        mesh=mesh,
        compiler_params=pltpu.CompilerParams(needs_layout_passes=False),
        out_type=jax.ShapeDtypeStruct((n_chunks, _CH * _LANES), jnp.float32),
        scratch_types=(
            [pltpu.VMEM((_LANES * wl,), jnp.float32) for wl in level_w]
            + [pltpu.VMEM((_G * _LANES,), jnp.float32),
               pltpu.VMEM((_G * _LANES,), jnp.float32),
               pltpu.VMEM((_CH * _LANES,), jnp.float32)]
        ),
    )
    def sampler(p0_hbm, p1_hbm, p2_hbm, p3_hbm, crd_hbm, sig_hbm, out_hbm,
                p0_v, p1_v, p2_v, p3_v, c_v, s_v, o_v):
        wid = lax.axis_index("s") * 2 + lax.axis_index("c")
        base = wid * chunks_per_w
        lane = lax.broadcasted_iota(jnp.int32, (_LANES,), 0)
        pyr_vs = [p0_v, p1_v, p2_v, p3_v]
        lane_off = [lane * wl for wl in level_w]

        def body(i, carry):
            t = base + i
            pltpu.sync_copy(p0_hbm.at[t], p0_v)
            pltpu.sync_copy(p1_hbm.at[t], p1_v)
            pltpu.sync_copy(p2_hbm.at[t], p2_v)
            pltpu.sync_copy(p3_hbm.at[t], p3_v)
            pltpu.sync_copy(crd_hbm.at[t], c_v)
            pltpu.sync_copy(sig_hbm.at[t], s_v)
            for g in range(_G):
                cg = c_v[pl.ds(g * _LANES, _LANES)]
                sg = s_v[pl.ds(g * _LANES, _LANES)]
                for s in range(_SAMPLES):
                    x = cg + jnp.float32(s - _SAMPLES // 2) * sg
                    for l in range(_NLVL):
                        wl = level_w[l]
                        xi = x * jnp.float32(1.0 / (2 ** l)) if l else x
                        xt = xi.astype(jnp.int32)
                        f = jnp.where(xi < xt.astype(jnp.float32), xt - 1, xt)
                        w1 = xi - f.astype(jnp.float32)
                        w0 = jnp.float32(1.0) - w1
                        i1 = f + 1
                        ok0 = (f >= 0) & (f <= wl - 1)
                        ok1 = (i1 >= 0) & (i1 <= wl - 1)
                        i0c = jnp.clip(f, 0, wl - 1) + lane_off[l]
                        i1c = jnp.clip(i1, 0, wl - 1) + lane_off[l]
                        v0 = plsc.load_gather(pyr_vs[l], [i0c])
                        v1 = plsc.load_gather(pyr_vs[l], [i1c])
                        v0 = jnp.where(ok0, v0, jnp.float32(0.0))
                        v1 = jnp.where(ok1, v1, jnp.float32(0.0))
                        ch = l * _GS + g * _SAMPLES + s
                        o_v[pl.ds(ch * _LANES, _LANES)] = w0 * v0 + w1 * v1
            pltpu.sync_copy(o_v, out_hbm.at[t])
            return carry

        lax.fori_loop(0, chunks_per_w, body, 0)

    return sampler


def kernel(fmap1, fmap2, coords, sigma):
    b, c, h, w = fmap1.shape
    g = coords.shape[1]
    n = b * h * w
    n_chunks = n // _LANES
    level_w = [w // (2 ** l) for l in range(_NLVL)]

    f1t = jnp.transpose(fmap1, (0, 2, 1, 3))
    f2t = jnp.transpose(fmap2, (0, 2, 1, 3))
    pyr = _corr_pyramid(f1t, f2t)
    pyr_c = [p.reshape(n_chunks, _LANES * wl) for p, wl in zip(pyr, level_w)]

    def to_chunked(a):
        return (a.transpose(0, 2, 3, 1)
                 .reshape(b, h, w // _LANES, _LANES, g)
                 .transpose(0, 1, 2, 4, 3)
                 .reshape(n_chunks, g * _LANES))

    crd = to_chunked(coords)
    sig = to_chunked(sigma)

    sampler = _make_sc_sampler(n_chunks, level_w)
    buf = sampler(*pyr_c, crd, sig)

    out = (buf.reshape(b, h, w // _LANES, _CH, _LANES)
              .transpose(0, 3, 1, 2, 4)
              .reshape(b, _CH, h, w))
    return out


# speedup vs baseline: 2.1756x; 1.3152x over previous
import functools
import math

import jax
import jax.numpy as jnp
from jax import lax
from jax.experimental import pallas as pl
from jax.experimental.pallas import tpu as pltpu
from jax.experimental.pallas import tpu_sc as plsc

_SAMPLES = 9
_NLVL = 4
_G = 4
_GS = _G * _SAMPLES
_CH = _NLVL * _GS
_LANES = 16


def _corr_pyr_body(f1_ref, f2_ref, o0_ref, o1_ref, o2_ref, o3_ref):
    f1 = f1_ref[0, 0]
    f2 = f2_ref[0, 0]
    c = f1.shape[0]
    scale = jnp.float32(1.0 / math.sqrt(float(c)))
    corr = lax.dot_general(f1, f2, (((0,), (0,)), ((), ())),
                           preferred_element_type=jnp.float32) * scale
    outs = [o0_ref, o1_ref, o2_ref, o3_ref]
    cur = corr
    outs[0][0, 0] = cur
    for lvl in range(1, _NLVL):
        wl = cur.shape[1]
        wn = wl // 2
        r = lax.broadcasted_iota(jnp.int32, (wl, wn), 0)
        cc = lax.broadcasted_iota(jnp.int32, (wl, wn), 1)
        pool = jnp.where(r // 2 == cc, jnp.float32(0.5), jnp.float32(0.0))
        cur = lax.dot_general(cur, pool, (((1,), (0,)), ((), ())),
                              preferred_element_type=jnp.float32)
        outs[lvl][0, 0] = cur


def _corr_pyramid(f1t, f2t):
    b, h, c, w = f1t.shape
    level_w = [w // (2 ** l) for l in range(_NLVL)]
    return pl.pallas_call(
        _corr_pyr_body,
        grid=(b, h),
        in_specs=[pl.BlockSpec((1, 1, c, w), lambda i, j: (i, j, 0, 0))] * 2,
        out_specs=[pl.BlockSpec((1, 1, w, wl), lambda i, j: (i, j, 0, 0))
                   for wl in level_w],
        out_shape=[jax.ShapeDtypeStruct((b, h, w, wl), jnp.float32)
                   for wl in level_w],
    )(f1t, f2t)


def _make_sc_sampler(n_chunks, level_w):
    mesh = plsc.VectorSubcoreMesh(core_axis_name="c", subcore_axis_name="s")
    n_workers = 32
    chunks_per_w = n_chunks // n_workers

    @functools.partial(
        pl.kernel,
        mesh=mesh,
        compiler_params=pltpu.CompilerParams(needs_layout_passes=False),
        out_type=jax.ShapeDtypeStruct((n_chunks, _CH * _LANES), jnp.float32),
        scratch_types=(
            [pltpu.VMEM((_LANES * wl,), jnp.float32)
             for wl in level_w for _ in range(2)]
            + [pltpu.VMEM((_G * _LANES,), jnp.float32) for _ in range(4)]
            + [pltpu.VMEM((_CH * _LANES,), jnp.float32) for _ in range(2)]
            + [pltpu.SemaphoreType.DMA for _ in range(4)]
        ),
    )
    def sampler(p0_hbm, p1_hbm, p2_hbm, p3_hbm, crd_hbm, sig_hbm, out_hbm,
                p0a, p0b, p1a, p1b, p2a, p2b, p3a, p3b,
                ca, cb, sa, sb, oa, ob, sem_a, sem_b, sem_oa, sem_ob):
        wid = lax.axis_index("s") * 2 + lax.axis_index("c")
        base = wid * chunks_per_w
        lane = lax.broadcasted_iota(jnp.int32, (_LANES,), 0)
        pyr_vs = [[p0a, p1a, p2a, p3a], [p0b, p1b, p2b, p3b]]
        c_vs = [ca, cb]
        s_vs = [sa, sb]
        o_vs = [oa, ob]
        sem_in = [sem_a, sem_b]
        sem_out = [sem_oa, sem_ob]
        pyr_hbms = [p0_hbm, p1_hbm, p2_hbm, p3_hbm]
        lane_off = [lane * wl for wl in level_w]

        def in_copies(t, p):
            for l in range(_NLVL):
                yield pltpu.make_async_copy(pyr_hbms[l].at[t], pyr_vs[p][l],
                                            sem_in[p])
            yield pltpu.make_async_copy(crd_hbm.at[t], c_vs[p], sem_in[p])
            yield pltpu.make_async_copy(sig_hbm.at[t], s_vs[p], sem_in[p])

        def start_in(t, p):
            for cp in in_copies(t, p):
                cp.start()

        def wait_in(t, p):
            for cp in in_copies(t, p):
                cp.wait()

        def compute(t, p):
            for g in range(_G):
                cg = c_vs[p][pl.ds(g * _LANES, _LANES)]
                sg = s_vs[p][pl.ds(g * _LANES, _LANES)]
                for s in range(_SAMPLES):
                    x = cg + jnp.float32(s - _SAMPLES // 2) * sg
                    xt = x.astype(jnp.int32)
                    f0 = jnp.where(x < xt.astype(jnp.float32), xt - 1, xt)
                    for l in range(_NLVL):
                        wl = level_w[l]
                        xi = x * jnp.float32(1.0 / (2 ** l)) if l else x
                        f = lax.shift_right_arithmetic(f0, l) if l else f0
                        w1 = xi - f.astype(jnp.float32)
                        w0 = jnp.float32(1.0) - w1
                        i1 = f + 1
                        ok0 = (f >= 0) & (f <= wl - 1)
                        ok1 = (i1 >= 0) & (i1 <= wl - 1)
                        i0c = jnp.clip(f, 0, wl - 1) + lane_off[l]
                        i1c = jnp.clip(i1, 0, wl - 1) + lane_off[l]
                        v0 = plsc.load_gather(pyr_vs[p][l], [i0c])
                        v1 = plsc.load_gather(pyr_vs[p][l], [i1c])
                        v0 = jnp.where(ok0, v0, jnp.float32(0.0))
                        v1 = jnp.where(ok1, v1, jnp.float32(0.0))
                        ch = l * _GS + g * _SAMPLES + s
                        o_vs[p][pl.ds(ch * _LANES, _LANES)] = w0 * v0 + w1 * v1

        start_in(base, 0)

        def body(i2, carry):
            for par in range(2):
                k = i2 * 2 + par
                t = base + k
                wait_in(t, par)
                if par == 0:
                    start_in(t + 1, 1)
                else:
                    @pl.when(i2 < chunks_per_w // 2 - 1)
                    def _start_next():
                        start_in(t + 1, 0)

                @pl.when(i2 >= 1)
                def _wait_prev_out():
                    pltpu.make_async_copy(o_vs[par], out_hbm.at[t],
                                          sem_out[par]).wait()

                compute(t, par)
                pltpu.make_async_copy(o_vs[par], out_hbm.at[t],
                                      sem_out[par]).start()
            return carry

        lax.fori_loop(0, chunks_per_w // 2, body, 0)
        last = base + chunks_per_w - 1
        pltpu.make_async_copy(o_vs[0], out_hbm.at[last - 1], sem_out[0]).wait()
        pltpu.make_async_copy(o_vs[1], out_hbm.at[last], sem_out[1]).wait()

    return sampler


def kernel(fmap1, fmap2, coords, sigma):
    b, c, h, w = fmap1.shape
    g = coords.shape[1]
    n = b * h * w
    n_chunks = n // _LANES
    level_w = [w // (2 ** l) for l in range(_NLVL)]

    f1t = jnp.transpose(fmap1, (0, 2, 1, 3))
    f2t = jnp.transpose(fmap2, (0, 2, 1, 3))
    pyr = _corr_pyramid(f1t, f2t)
    pyr_c = [p.reshape(n_chunks, _LANES * wl) for p, wl in zip(pyr, level_w)]

    def to_chunked(a):
        return (a.transpose(0, 2, 3, 1)
                 .reshape(b, h, w // _LANES, _LANES, g)
                 .transpose(0, 1, 2, 4, 3)
                 .reshape(n_chunks, g * _LANES))

    crd = to_chunked(coords)
    sig = to_chunked(sigma)

    sampler = _make_sc_sampler(n_chunks, level_w)
    buf = sampler(*pyr_c, crd, sig)

    out = (buf.reshape(b, h, w // _LANES, _CH, _LANES)
              .transpose(0, 3, 1, 2, 4)
              .reshape(b, _CH, h, w))
    return out


# speedup vs baseline: 2.6814x; 1.2325x over previous
import functools
import math

import jax
import jax.numpy as jnp
from jax import lax
from jax.experimental import pallas as pl
from jax.experimental.pallas import tpu as pltpu
from jax.experimental.pallas import tpu_sc as plsc

_SAMPLES = 9
_NLVL = 4
_G = 4
_GS = _G * _SAMPLES
_CH = _NLVL * _GS
_LANES = 16
_HB = 8


def _pool_matrix(wl, wn, scale):
    r = lax.broadcasted_iota(jnp.int32, (wl, wn), 0)
    c = lax.broadcasted_iota(jnp.int32, (wl, wn), 1)
    return jnp.where(r // 2 == c, jnp.float32(scale), jnp.float32(0.0))


def _corr_pyr_body(f1_ref, f2_ref, o0_ref, o1_ref, o23_ref):
    c = f1_ref.shape[1]
    scale = jnp.float32(1.0 / math.sqrt(float(c)))
    p1 = _pool_matrix(256, 128, 0.5)
    r = lax.broadcasted_iota(jnp.int32, (128, 128), 0)
    cc = lax.broadcasted_iota(jnp.int32, (128, 128), 1)
    q = jnp.where(
        (cc < 64) & (r // 2 == cc), jnp.float32(0.5),
        jnp.where((cc >= 64) & (cc < 96) & (r // 4 == cc - 64),
                  jnp.float32(0.25), jnp.float32(0.0)))
    for hh in range(_HB):
        f1 = f1_ref[0, :, hh, :]
        f2 = f2_ref[0, :, hh, :]
        corr = lax.dot_general(f1, f2, (((0,), (0,)), ((), ())),
                               preferred_element_type=jnp.float32) * scale
        o0_ref[0, hh] = jnp.concatenate([corr[:, :128], corr[:, 128:]], axis=0)
        l1 = lax.dot_general(corr, p1, (((1,), (0,)), ((), ())),
                             preferred_element_type=jnp.float32)
        o1_ref[0, hh] = l1
        o23_ref[0, hh] = lax.dot_general(l1, q, (((1,), (0,)), ((), ())),
                                         preferred_element_type=jnp.float32)


def _corr_pyramid(f1, f2):
    b, c, h, w = f1.shape
    shapes = [(b, h, 2 * w, 128), (b, h, w, 128), (b, h, w, 128)]
    return pl.pallas_call(
        _corr_pyr_body,
        grid=(b, h // _HB),
        in_specs=[pl.BlockSpec((1, c, _HB, w), lambda i, j: (i, 0, j, 0))] * 2,
        out_specs=[pl.BlockSpec((1, _HB, s[2], 128), lambda i, j: (i, j, 0, 0))
                   for s in shapes],
        out_shape=[jax.ShapeDtypeStruct(s, jnp.float32) for s in shapes],
    )(f1, f2)


def _make_sc_sampler(n_chunks, w):
    level_w = [w // (2 ** l) for l in range(_NLVL)]
    mesh = plsc.VectorSubcoreMesh(core_axis_name="c", subcore_axis_name="s")
    n_workers = 32
    chunks_per_w = n_chunks // n_workers

    @functools.partial(
        pl.kernel,
        mesh=mesh,
        compiler_params=pltpu.CompilerParams(needs_layout_passes=False),
        out_type=jax.ShapeDtypeStruct((n_chunks, _CH * _LANES), jnp.float32),
        scratch_types=(
            [pltpu.VMEM((4096,), jnp.float32) for _ in range(2)]
            + [pltpu.VMEM((2048,), jnp.float32) for _ in range(4)]
            + [pltpu.VMEM((_G * _LANES,), jnp.float32) for _ in range(4)]
            + [pltpu.VMEM((_CH * _LANES,), jnp.float32) for _ in range(2)]
            + [pltpu.SemaphoreType.DMA for _ in range(4)]
        ),
    )
    def sampler(p0_hbm, p1_hbm, p23_hbm, crd_hbm, sig_hbm, out_hbm,
                p0a, p0b, p1a, p1b, p23a, p23b,
                ca, cb, sa, sb, oa, ob, sem_a, sem_b, sem_oa, sem_ob):
        wid = lax.axis_index("s") * 2 + lax.axis_index("c")
        base = wid * chunks_per_w
        lane = lax.broadcasted_iota(jnp.int32, (_LANES,), 0)
        lane128 = lane * 128
        bufs = [[p0a, p1a, p23a, p23a], [p0b, p1b, p23b, p23b]]
        c_vs = [ca, cb]
        s_vs = [sa, sb]
        o_vs = [oa, ob]
        sem_in = [sem_a, sem_b]
        sem_out = [sem_oa, sem_ob]

        def in_copies(t, p):
            bh = lax.shift_right_logical(t, 4)
            ra = t + (bh << 4)
            p0_v = bufs[p][0]
            yield pltpu.make_async_copy(p0_hbm.at[ra],
                                        p0_v.at[pl.ds(0, 2048)], sem_in[p])
            yield pltpu.make_async_copy(p0_hbm.at[ra + 16],
                                        p0_v.at[pl.ds(2048, 2048)], sem_in[p])
            yield pltpu.make_async_copy(p1_hbm.at[t], bufs[p][1], sem_in[p])
            yield pltpu.make_async_copy(p23_hbm.at[t], bufs[p][2], sem_in[p])
            yield pltpu.make_async_copy(crd_hbm.at[t], c_vs[p], sem_in[p])
            yield pltpu.make_async_copy(sig_hbm.at[t], s_vs[p], sem_in[p])

        def start_in(t, p):
            for cp in in_copies(t, p):
                cp.start()

        def wait_in(t, p):
            for cp in in_copies(t, p):
                cp.wait()

        def compute(p):
            for g in range(_G):
                cg = c_vs[p][pl.ds(g * _LANES, _LANES)]
                sg = s_vs[p][pl.ds(g * _LANES, _LANES)]
                for s in range(_SAMPLES):
                    x = cg + jnp.float32(s - _SAMPLES // 2) * sg
                    xt = x.astype(jnp.int32)
                    f0 = jnp.where(x < xt.astype(jnp.float32), xt - 1, xt)
                    for l in range(_NLVL):
                        wl = level_w[l]
                        xi = x * jnp.float32(1.0 / (2 ** l)) if l else x
                        f = lax.shift_right_arithmetic(f0, l) if l else f0
                        w1 = xi - f.astype(jnp.float32)
                        w0 = jnp.float32(1.0) - w1
                        i1 = f + 1
                        c0 = jnp.clip(f, 0, wl - 1)
                        c1 = jnp.clip(i1, 0, wl - 1)
                        if l == 0:
                            off0 = lane128 + (c0 & 127) + ((c0 & 128) << 4)
                            off1 = lane128 + (c1 & 127) + ((c1 & 128) << 4)
                        elif l == 3:
                            off0 = lane128 + (c0 + 64)
                            off1 = lane128 + (c1 + 64)
                        else:
                            off0 = lane128 + c0
                            off1 = lane128 + c1
                        v0 = plsc.load_gather(bufs[p][l], [off0])
                        v1 = plsc.load_gather(bufs[p][l], [off1])
                        v0 = jnp.where(f == c0, v0, jnp.float32(0.0))
                        v1 = jnp.where(i1 == c1, v1, jnp.float32(0.0))
                        ch = l * _GS + g * _SAMPLES + s
                        o_vs[p][pl.ds(ch * _LANES, _LANES)] = w0 * v0 + w1 * v1

        start_in(base, 0)

        def body(i2, carry):
            for par in range(2):
                k = i2 * 2 + par
                t = base + k
                wait_in(t, par)
                if par == 0:
                    start_in(t + 1, 1)
                else:
                    @pl.when(i2 < chunks_per_w // 2 - 1)
                    def _start_next():
                        start_in(t + 1, 0)

                @pl.when(i2 >= 1)
                def _wait_prev_out():
                    pltpu.make_async_copy(o_vs[par], out_hbm.at[t],
                                          sem_out[par]).wait()

                compute(par)
                pltpu.make_async_copy(o_vs[par], out_hbm.at[t],
                                      sem_out[par]).start()
            return carry

        lax.fori_loop(0, chunks_per_w // 2, body, 0)
        last = base + chunks_per_w - 1
        pltpu.make_async_copy(o_vs[0], out_hbm.at[last - 1], sem_out[0]).wait()
        pltpu.make_async_copy(o_vs[1], out_hbm.at[last], sem_out[1]).wait()

    return sampler


def kernel(fmap1, fmap2, coords, sigma):
    b, c, h, w = fmap1.shape
    g = coords.shape[1]
    n = b * h * w
    n_chunks = n // _LANES

    o0, o1, o23 = _corr_pyramid(fmap1, fmap2)
    p0 = o0.reshape(b * h * 32, 16 * 128)
    p1 = o1.reshape(n_chunks, 16 * 128)
    p23 = o23.reshape(n_chunks, 16 * 128)

    def to_chunked(a):
        return (a.transpose(0, 2, 3, 1)
                 .reshape(b, h, w // _LANES, _LANES, g)
                 .transpose(0, 1, 2, 4, 3)
                 .reshape(n_chunks, g * _LANES))

    crd = to_chunked(coords)
    sig = to_chunked(sigma)

    sampler = _make_sc_sampler(n_chunks, w)
    buf = sampler(p0, p1, p23, crd, sig)

    out = (buf.reshape(b, h, w // _LANES, _CH, _LANES)
              .transpose(0, 3, 1, 2, 4)
              .reshape(b, _CH, h, w))
    return out


# speedup vs baseline: 3.1352x; 1.1692x over previous
import functools
import math

import jax
import jax.numpy as jnp
from jax import lax
from jax.experimental import pallas as pl
from jax.experimental.pallas import tpu as pltpu
from jax.experimental.pallas import tpu_sc as plsc

_SAMPLES = 9
_NLVL = 4
_G = 4
_GS = _G * _SAMPLES
_CH = _NLVL * _GS
_LANES = 16
_HB = 8


def _pool_matrix(wl, wn, scale):
    r = lax.broadcasted_iota(jnp.int32, (wl, wn), 0)
    c = lax.broadcasted_iota(jnp.int32, (wl, wn), 1)
    return jnp.where(r // 2 == c, jnp.float32(scale), jnp.float32(0.0))


def _corr_pyr_body(f1_ref, f2_ref, o0_ref, o1_ref, o23_ref):
    c = f1_ref.shape[1]
    scale = jnp.float32(1.0 / math.sqrt(float(c)))
    p1 = _pool_matrix(256, 128, 0.5)
    r = lax.broadcasted_iota(jnp.int32, (128, 128), 0)
    cc = lax.broadcasted_iota(jnp.int32, (128, 128), 1)
    q = jnp.where(
        (cc < 64) & (r // 2 == cc), jnp.float32(0.5),
        jnp.where((cc >= 64) & (cc < 96) & (r // 4 == cc - 64),
                  jnp.float32(0.25), jnp.float32(0.0)))
    for hh in range(_HB):
        f1 = f1_ref[0, :, hh, :]
        f2 = f2_ref[0, :, hh, :]
        corr = lax.dot_general(f1, f2, (((0,), (0,)), ((), ())),
                               preferred_element_type=jnp.float32) * scale
        o0_ref[0, hh] = jnp.concatenate([corr[:, :128], corr[:, 128:]], axis=0)
        l1 = lax.dot_general(corr, p1, (((1,), (0,)), ((), ())),
                             preferred_element_type=jnp.float32)
        o1_ref[0, hh] = l1
        o23_ref[0, hh] = lax.dot_general(l1, q, (((1,), (0,)), ((), ())),
                                         preferred_element_type=jnp.float32)


def _corr_pyramid(f1, f2):
    b, c, h, w = f1.shape
    shapes = [(b, h, 2 * w, 128), (b, h, w, 128), (b, h, w, 128)]
    return pl.pallas_call(
        _corr_pyr_body,
        grid=(b, h // _HB),
        in_specs=[pl.BlockSpec((1, c, _HB, w), lambda i, j: (i, 0, j, 0))] * 2,
        out_specs=[pl.BlockSpec((1, _HB, s[2], 128), lambda i, j: (i, j, 0, 0))
                   for s in shapes],
        out_shape=[jax.ShapeDtypeStruct(s, jnp.float32) for s in shapes],
    )(f1, f2)


def _make_sc_sampler(b_sz, h_sz, w):
    level_w = [w // (2 ** l) for l in range(_NLVL)]
    n_chunks = b_sz * h_sz * (w // _LANES)
    mesh = plsc.VectorSubcoreMesh(core_axis_name="c", subcore_axis_name="s")
    n_workers = 32
    chunks_per_w = n_chunks // n_workers
    wc = w // _LANES

    @functools.partial(
        pl.kernel,
        mesh=mesh,
        compiler_params=pltpu.CompilerParams(needs_layout_passes=False),
        out_type=jax.ShapeDtypeStruct((n_chunks, _CH * _LANES), jnp.float32),
        scratch_types=(
            [pltpu.VMEM((2 * _LANES, 128), jnp.float32) for _ in range(2)]
            + [pltpu.VMEM((_LANES, 128), jnp.float32) for _ in range(4)]
            + [pltpu.VMEM((_G * _LANES,), jnp.float32) for _ in range(4)]
            + [pltpu.VMEM((_CH * _LANES,), jnp.float32) for _ in range(2)]
            + [pltpu.SemaphoreType.DMA for _ in range(4)]
        ),
    )
    def sampler(p0_hbm, p1_hbm, p23_hbm, crd_hbm, sig_hbm, out_hbm,
                p0a, p0b, p1a, p1b, p23a, p23b,
                ca, cb, sa, sb, oa, ob, sem_a, sem_b, sem_oa, sem_ob):
        wid = lax.axis_index("s") * 2 + lax.axis_index("c")
        base = wid * chunks_per_w
        lane = lax.broadcasted_iota(jnp.int32, (_LANES,), 0)
        bufs = [[p0a, p1a, p23a, p23a], [p0b, p1b, p23b, p23b]]
        c_vs = [ca, cb]
        s_vs = [sa, sb]
        o_vs = [oa, ob]
        sem_in = [sem_a, sem_b]
        sem_out = [sem_oa, sem_ob]

        def decode(t):
            bi = lax.shift_right_logical(t, 10)
            hi = lax.shift_right_logical(t, 4) & (h_sz - 1)
            s = t & (wc - 1)
            return bi, hi, s

        def in_copies(t, p):
            bi, hi, s = decode(t)
            r0 = s * _LANES
            w0 = s * _LANES
            p0_v = bufs[p][0]
            yield pltpu.make_async_copy(
                p0_hbm.at[bi, hi, pl.ds(r0, _LANES)],
                p0_v.at[pl.ds(0, _LANES)], sem_in[p])
            yield pltpu.make_async_copy(
                p0_hbm.at[bi, hi, pl.ds(w + r0, _LANES)],
                p0_v.at[pl.ds(_LANES, _LANES)], sem_in[p])
            yield pltpu.make_async_copy(
                p1_hbm.at[bi, hi, pl.ds(r0, _LANES)], bufs[p][1], sem_in[p])
            yield pltpu.make_async_copy(
                p23_hbm.at[bi, hi, pl.ds(r0, _LANES)], bufs[p][2], sem_in[p])
            yield pltpu.make_async_copy(crd_hbm.at[t], c_vs[p], sem_in[p])
            yield pltpu.make_async_copy(sig_hbm.at[t], s_vs[p], sem_in[p])

        def start_in(t, p):
            for cp in in_copies(t, p):
                cp.start()

        def wait_in(t, p):
            for cp in in_copies(t, p):
                cp.wait()

        def out_copy(t, p):
            return pltpu.make_async_copy(o_vs[p], out_hbm.at[t], sem_out[p])

        def compute(p):
            for g in range(_G):
                cg = c_vs[p][pl.ds(g * _LANES, _LANES)]
                sg = s_vs[p][pl.ds(g * _LANES, _LANES)]
                for s in range(_SAMPLES):
                    x = cg + jnp.float32(s - _SAMPLES // 2) * sg
                    xt = x.astype(jnp.int32)
                    f0 = jnp.where(x < xt.astype(jnp.float32), xt - 1, xt)
                    for l in range(_NLVL):
                        wl = level_w[l]
                        xi = x * jnp.float32(1.0 / (2 ** l)) if l else x
                        f = lax.shift_right_arithmetic(f0, l) if l else f0
                        w1 = xi - f.astype(jnp.float32)
                        w0 = jnp.float32(1.0) - w1
                        i1 = f + 1
                        c0 = jnp.clip(f, 0, wl - 1)
                        c1 = jnp.clip(i1, 0, wl - 1)
                        if l == 0:
                            r0v = lane + lax.shift_right_logical(c0 & 128, 3)
                            r1v = lane + lax.shift_right_logical(c1 & 128, 3)
                            v0 = plsc.load_gather(bufs[p][0], [r0v, c0 & 127])
                            v1 = plsc.load_gather(bufs[p][0], [r1v, c1 & 127])
                        elif l == 3:
                            v0 = plsc.load_gather(bufs[p][3], [lane, c0 + 64])
                            v1 = plsc.load_gather(bufs[p][3], [lane, c1 + 64])
                        else:
                            v0 = plsc.load_gather(bufs[p][l], [lane, c0])
                            v1 = plsc.load_gather(bufs[p][l], [lane, c1])
                        v0 = jnp.where(f == c0, v0, jnp.float32(0.0))
                        v1 = jnp.where(i1 == c1, v1, jnp.float32(0.0))
                        ch = l * _GS + g * _SAMPLES + s
                        o_vs[p][pl.ds(ch * _LANES, _LANES)] = w0 * v0 + w1 * v1

        start_in(base, 0)

        def body(i2, carry):
            for par in range(2):
                k = i2 * 2 + par
                t = base + k
                wait_in(t, par)
                if par == 0:
                    start_in(t + 1, 1)
                else:
                    @pl.when(i2 < chunks_per_w // 2 - 1)
                    def _start_next():
                        start_in(t + 1, 0)

                @pl.when(i2 >= 1)
                def _wait_prev_out():
                    out_copy(t, par).wait()

                compute(par)
                out_copy(t, par).start()
            return carry

        lax.fori_loop(0, chunks_per_w // 2, body, 0)
        last = base + chunks_per_w - 1
        out_copy(last - 1, 0).wait()
        out_copy(last, 1).wait()

    return sampler


def kernel(fmap1, fmap2, coords, sigma):
    b, c, h, w = fmap1.shape
    g = coords.shape[1]
    n_chunks = b * h * w // _LANES
    o0, o1, o23 = _corr_pyramid(fmap1, fmap2)

    def to_chunked(a):
        return (a.transpose(0, 2, 3, 1)
                 .reshape(b, h, w // _LANES, _LANES, g)
                 .transpose(0, 1, 2, 4, 3)
                 .reshape(n_chunks, g * _LANES))

    sampler = _make_sc_sampler(b, h, w)
    buf = sampler(o0, o1, o23, to_chunked(coords), to_chunked(sigma))
    out = (buf.reshape(b, h, w // _LANES, _CH, _LANES)
              .transpose(0, 3, 1, 2, 4)
              .reshape(b, _CH, h, w))
    return out


# speedup vs baseline: 5.0398x; 1.6075x over previous
import functools
import math

import jax
import jax.numpy as jnp
from jax import lax
from jax.experimental import pallas as pl
from jax.experimental.pallas import tpu as pltpu
from jax.experimental.pallas import tpu_sc as plsc

_SAMPLES = 9
_NLVL = 4
_G = 4
_GS = _G * _SAMPLES
_CH = _NLVL * _GS
_LANES = 16
_HB = 8


def _pool_matrix(wl, wn, scale):
    r = lax.broadcasted_iota(jnp.int32, (wl, wn), 0)
    c = lax.broadcasted_iota(jnp.int32, (wl, wn), 1)
    return jnp.where(r // 2 == c, jnp.float32(scale), jnp.float32(0.0))


def _corr_pyr_body(f1_ref, f2_ref, o0_ref, o1_ref, o23_ref):
    c = f1_ref.shape[1]
    scale = jnp.float32(1.0 / math.sqrt(float(c)))
    p1 = _pool_matrix(256, 128, 0.5)
    r = lax.broadcasted_iota(jnp.int32, (128, 128), 0)
    cc = lax.broadcasted_iota(jnp.int32, (128, 128), 1)
    q = jnp.where(
        (cc < 64) & (r // 2 == cc), jnp.float32(0.5),
        jnp.where((cc >= 64) & (cc < 96) & (r // 4 == cc - 64),
                  jnp.float32(0.25), jnp.float32(0.0)))
    for hh in range(_HB):
        f1 = f1_ref[0, :, hh, :]
        f2 = f2_ref[0, :, hh, :]
        corr = lax.dot_general(f1, f2, (((0,), (0,)), ((), ())),
                               preferred_element_type=jnp.float32) * scale
        o0_ref[0, hh] = jnp.concatenate([corr[:, :128], corr[:, 128:]], axis=0)
        l1 = lax.dot_general(corr, p1, (((1,), (0,)), ((), ())),
                             preferred_element_type=jnp.float32)
        o1_ref[0, hh] = l1
        o23_ref[0, hh] = lax.dot_general(l1, q, (((1,), (0,)), ((), ())),
                                         preferred_element_type=jnp.float32)


def _corr_pyramid(f1, f2):
    b, c, h, w = f1.shape
    shapes = [(b, h, 2 * w, 128), (b, h, w, 128), (b, h, w, 128)]
    return pl.pallas_call(
        _corr_pyr_body,
        grid=(b, h // _HB),
        in_specs=[pl.BlockSpec((1, c, _HB, w), lambda i, j: (i, 0, j, 0))] * 2,
        out_specs=[pl.BlockSpec((1, _HB, s[2], 128), lambda i, j: (i, j, 0, 0))
                   for s in shapes],
        out_shape=[jax.ShapeDtypeStruct(s, jnp.float32) for s in shapes],
    )(f1, f2)


def _make_sc_sampler(b_sz, h_sz, w):
    level_w = [w // (2 ** l) for l in range(_NLVL)]
    n_chunks = b_sz * h_sz * (w // _LANES)
    mesh = plsc.VectorSubcoreMesh(core_axis_name="c", subcore_axis_name="s")
    wc = w // _LANES
    hcw = _CH // 2

    @functools.partial(
        pl.kernel,
        mesh=mesh,
        compiler_params=pltpu.CompilerParams(needs_layout_passes=False),
        out_type=jax.ShapeDtypeStruct((b_sz, _CH, h_sz, w), jnp.float32),
        scratch_types=(
            [pltpu.VMEM((2 * _LANES, 128), jnp.float32) for _ in range(2)]
            + [pltpu.VMEM((_LANES, 128), jnp.float32) for _ in range(4)]
            + [pltpu.VMEM((_G * _LANES,), jnp.float32) for _ in range(4)]
            + [pltpu.VMEM((hcw, 8, 128), jnp.float32)]
            + [pltpu.SemaphoreType.DMA for _ in range(3)]
        ),
    )
    def sampler(p0_hbm, p1_hbm, p23_hbm, crd_hbm, sig_hbm, out_hbm,
                p0a, p0b, p1a, p1b, p23a, p23b,
                ca, cb, sa, sb, o_v, sem_a, sem_b, sem_o):
        wid = lax.axis_index("s") * 2 + lax.axis_index("c")
        bi = lax.shift_right_logical(wid, 4)
        hb = lax.shift_right_logical(wid, 1) & 7
        wq = wid & 1
        lane = lax.broadcasted_iota(jnp.int32, (_LANES,), 0)
        bufs = [[p0a, p1a, p23a, p23a], [p0b, p1b, p23b, p23b]]
        c_vs = [ca, cb]
        s_vs = [sa, sb]
        sem_in = [sem_a, sem_b]

        def chunk_coords(k):
            hi = hb * 8 + lax.shift_right_logical(k, 3)
            s = wq * 8 + (k & 7)
            t = (bi * h_sz + hi) * wc + s
            return hi, s, t

        def in_copies(k, p, half):
            hi, s, t = chunk_coords(k)
            r0 = s * _LANES
            if half == 0:
                p0_v = bufs[p][0]
                yield pltpu.make_async_copy(
                    p0_hbm.at[bi, hi, pl.ds(r0, _LANES)],
                    p0_v.at[pl.ds(0, _LANES)], sem_in[p])
                yield pltpu.make_async_copy(
                    p0_hbm.at[bi, hi, pl.ds(w + r0, _LANES)],
                    p0_v.at[pl.ds(_LANES, _LANES)], sem_in[p])
                yield pltpu.make_async_copy(
                    p1_hbm.at[bi, hi, pl.ds(r0, _LANES)], bufs[p][1],
                    sem_in[p])
            else:
                yield pltpu.make_async_copy(
                    p23_hbm.at[bi, hi, pl.ds(r0, _LANES)], bufs[p][2],
                    sem_in[p])
            yield pltpu.make_async_copy(crd_hbm.at[t], c_vs[p], sem_in[p])
            yield pltpu.make_async_copy(sig_hbm.at[t], s_vs[p], sem_in[p])

        def start_in(k, p, half):
            for cp in in_copies(k, p, half):
                cp.start()

        def wait_in(k, p, half):
            for cp in in_copies(k, p, half):
                cp.wait()

        def out_copy(half):
            return pltpu.make_async_copy(
                o_v,
                out_hbm.at[bi, pl.ds(half * hcw, hcw),
                           pl.ds(hb * 8, 8), pl.ds(wq * 128, 128)],
                sem_o)

        def compute(k, p, half):
            h_off = lax.shift_right_logical(k, 3)
            woff = (k & 7) * _LANES
            levels = (0, 1) if half == 0 else (2, 3)
            for g in range(_G):
                cg = c_vs[p][pl.ds(g * _LANES, _LANES)]
                sg = s_vs[p][pl.ds(g * _LANES, _LANES)]
                for s in range(_SAMPLES):
                    x = cg + jnp.float32(s - _SAMPLES // 2) * sg
                    xt = x.astype(jnp.int32)
                    f0 = jnp.where(x < xt.astype(jnp.float32), xt - 1, xt)
                    for l in levels:
                        wl = level_w[l]
                        xi = x * jnp.float32(1.0 / (2 ** l)) if l else x
                        f = lax.shift_right_arithmetic(f0, l) if l else f0
                        w1 = xi - f.astype(jnp.float32)
                        w0 = jnp.float32(1.0) - w1
                        i1 = f + 1
                        c0 = jnp.clip(f, 0, wl - 1)
                        c1 = jnp.clip(i1, 0, wl - 1)
                        if l == 0:
                            r0v = lane + lax.shift_right_logical(c0 & 128, 3)
                            r1v = lane + lax.shift_right_logical(c1 & 128, 3)
                            v0 = plsc.load_gather(bufs[p][0], [r0v, c0 & 127])
                            v1 = plsc.load_gather(bufs[p][0], [r1v, c1 & 127])
                        elif l == 3:
                            v0 = plsc.load_gather(bufs[p][3], [lane, c0 + 64])
                            v1 = plsc.load_gather(bufs[p][3], [lane, c1 + 64])
                        else:
                            v0 = plsc.load_gather(bufs[p][l], [lane, c0])
                            v1 = plsc.load_gather(bufs[p][l], [lane, c1])
                        v0 = jnp.where(f == c0, v0, jnp.float32(0.0))
                        v1 = jnp.where(i1 == c1, v1, jnp.float32(0.0))
                        ch = l * _GS + g * _SAMPLES + s - half * hcw
                        o_v[ch, h_off, pl.ds(woff, _LANES)] = w0 * v0 + w1 * v1

        def run_pass(half, first):
            def body(k2, carry):
                for qp in range(2):
                    k = k2 * 2 + qp
                    wait_in(k, qp, half)
                    if qp == 1:
                        @pl.when(k2 < 31)
                        def _start_next():
                            start_in(k + 1, 0, half)
                    else:
                        start_in(k + 1, 1, half)
                    compute(k, qp, half)
                return carry

            if not first:
                out_copy(0).wait()
            lax.fori_loop(0, 32, body, 0)
            out_copy(half).start()

        start_in(0, 0, 0)
        run_pass(0, True)
        start_in(0, 0, 1)
        run_pass(1, False)
        out_copy(1).wait()

    return sampler


def kernel(fmap1, fmap2, coords, sigma):
    b, c, h, w = fmap1.shape
    g = coords.shape[1]
    n_chunks = b * h * w // _LANES
    o0, o1, o23 = _corr_pyramid(fmap1, fmap2)

    def to_chunked(a):
        return (a.transpose(0, 2, 3, 1)
                 .reshape(b, h, w // _LANES, _LANES, g)
                 .transpose(0, 1, 2, 4, 3)
                 .reshape(n_chunks, g * _LANES))

    sampler = _make_sc_sampler(b, h, w)
    return sampler(o0, o1, o23, to_chunked(coords), to_chunked(sigma))


# speedup vs baseline: 5.1099x; 1.0139x over previous
import functools
import math

import jax
import jax.numpy as jnp
from jax import lax
from jax.experimental import pallas as pl
from jax.experimental.pallas import tpu as pltpu
from jax.experimental.pallas import tpu_sc as plsc

_SAMPLES = 9
_NLVL = 4
_G = 4
_GS = _G * _SAMPLES
_CH = _NLVL * _GS
_LANES = 16
_HB = 8


def _pool_matrix(wl, wn, scale):
    r = lax.broadcasted_iota(jnp.int32, (wl, wn), 0)
    c = lax.broadcasted_iota(jnp.int32, (wl, wn), 1)
    return jnp.where(r // 2 == c, jnp.float32(scale), jnp.float32(0.0))


def _corr_pyr_body(f1_ref, f2_ref, o0_ref, o1_ref, o23_ref):
    c = f1_ref.shape[1]
    scale = jnp.float32(1.0 / math.sqrt(float(c)))
    p1 = _pool_matrix(256, 128, 0.5)
    r = lax.broadcasted_iota(jnp.int32, (128, 128), 0)
    cc = lax.broadcasted_iota(jnp.int32, (128, 128), 1)
    q = jnp.where(
        (cc < 64) & (r // 2 == cc), jnp.float32(0.5),
        jnp.where((cc >= 64) & (cc < 96) & (r // 4 == cc - 64),
                  jnp.float32(0.25), jnp.float32(0.0)))
    for hh in range(_HB):
        f1 = f1_ref[0, :, hh, :]
        f2 = f2_ref[0, :, hh, :]
        corr = lax.dot_general(f1, f2, (((0,), (0,)), ((), ())),
                               preferred_element_type=jnp.float32) * scale
        o0_ref[0, hh] = jnp.concatenate([corr[:, :128], corr[:, 128:]], axis=0)
        l1 = lax.dot_general(corr, p1, (((1,), (0,)), ((), ())),
                             preferred_element_type=jnp.float32)
        o1_ref[0, hh] = l1
        o23_ref[0, hh] = lax.dot_general(l1, q, (((1,), (0,)), ((), ())),
                                         preferred_element_type=jnp.float32)


def _corr_pyramid(f1, f2):
    b, c, h, w = f1.shape
    shapes = [(b, h, 2 * w, 128), (b, h, w, 128), (b, h, w, 128)]
    return pl.pallas_call(
        _corr_pyr_body,
        grid=(b, h // _HB),
        in_specs=[pl.BlockSpec((1, c, _HB, w), lambda i, j: (i, 0, j, 0))] * 2,
        out_specs=[pl.BlockSpec((1, _HB, s[2], 128), lambda i, j: (i, j, 0, 0))
                   for s in shapes],
        out_shape=[jax.ShapeDtypeStruct(s, jnp.float32) for s in shapes],
    )(f1, f2)


def _make_sc_sampler(b_sz, h_sz, w):
    level_w = [w // (2 ** l) for l in range(_NLVL)]
    n_chunks = b_sz * h_sz * (w // _LANES)
    mesh = plsc.VectorSubcoreMesh(core_axis_name="c", subcore_axis_name="s")
    wc = w // _LANES
    hcw = _CH // 2

    @functools.partial(
        pl.kernel,
        mesh=mesh,
        compiler_params=pltpu.CompilerParams(needs_layout_passes=False),
        out_type=jax.ShapeDtypeStruct((b_sz, _CH, h_sz, w), jnp.float32),
        scratch_types=(
            [pltpu.VMEM((2 * _LANES, 128), jnp.float32) for _ in range(2)]
            + [pltpu.VMEM((_LANES, 128), jnp.float32) for _ in range(4)]
            + [pltpu.VMEM((_G, 8, 128), jnp.float32) for _ in range(2)]
            + [pltpu.VMEM((hcw, 8, 128), jnp.float32)]
            + [pltpu.SemaphoreType.DMA for _ in range(3)]
        ),
    )
    def sampler(p0_hbm, p1_hbm, p23_hbm, crd_hbm, sig_hbm, out_hbm,
                p0a, p0b, p1a, p1b, p23a, p23b,
                c_v, s_v, o_v, sem_a, sem_b, sem_o):
        wid = lax.axis_index("s") * 2 + lax.axis_index("c")
        bi = lax.shift_right_logical(wid, 4)
        hb = lax.shift_right_logical(wid, 1) & 7
        wq = wid & 1
        lane = lax.broadcasted_iota(jnp.int32, (_LANES,), 0)
        bufs = [[p0a, p1a, p23a, p23a], [p0b, p1b, p23b, p23b]]
        sem_in = [sem_a, sem_b]

        def chunk_coords(k):
            hi = hb * 8 + lax.shift_right_logical(k, 3)
            s = wq * 8 + (k & 7)
            t = (bi * h_sz + hi) * wc + s
            return hi, s, t

        def in_copies(k, p, half):
            hi, s, t = chunk_coords(k)
            r0 = s * _LANES
            if half == 0:
                p0_v = bufs[p][0]
                yield pltpu.make_async_copy(
                    p0_hbm.at[bi, hi, pl.ds(r0, _LANES)],
                    p0_v.at[pl.ds(0, _LANES)], sem_in[p])
                yield pltpu.make_async_copy(
                    p0_hbm.at[bi, hi, pl.ds(w + r0, _LANES)],
                    p0_v.at[pl.ds(_LANES, _LANES)], sem_in[p])
                yield pltpu.make_async_copy(
                    p1_hbm.at[bi, hi, pl.ds(r0, _LANES)], bufs[p][1],
                    sem_in[p])
            else:
                yield pltpu.make_async_copy(
                    p23_hbm.at[bi, hi, pl.ds(r0, _LANES)], bufs[p][2],
                    sem_in[p])

        def coord_copies(sem):
            yield pltpu.make_async_copy(
                crd_hbm.at[bi, :, pl.ds(hb * 8, 8), pl.ds(wq * 128, 128)],
                c_v, sem)
            yield pltpu.make_async_copy(
                sig_hbm.at[bi, :, pl.ds(hb * 8, 8), pl.ds(wq * 128, 128)],
                s_v, sem)

        def start_in(k, p, half):
            for cp in in_copies(k, p, half):
                cp.start()

        def wait_in(k, p, half):
            for cp in in_copies(k, p, half):
                cp.wait()

        def out_copy(half):
            return pltpu.make_async_copy(
                o_v,
                out_hbm.at[bi, pl.ds(half * hcw, hcw),
                           pl.ds(hb * 8, 8), pl.ds(wq * 128, 128)],
                sem_o)

        def compute(k, p, half):
            h_off = lax.shift_right_logical(k, 3)
            woff = (k & 7) * _LANES
            levels = (0, 1) if half == 0 else (2, 3)
            for g in range(_G):
                cg = c_v[g, h_off, pl.ds(woff, _LANES)]
                sg = s_v[g, h_off, pl.ds(woff, _LANES)]
                for s in range(_SAMPLES):
                    x = cg + jnp.float32(s - _SAMPLES // 2) * sg
                    xt = x.astype(jnp.int32)
                    f0 = jnp.where(x < xt.astype(jnp.float32), xt - 1, xt)
                    for l in levels:
                        wl = level_w[l]
                        xi = x * jnp.float32(1.0 / (2 ** l)) if l else x
                        f = lax.shift_right_arithmetic(f0, l) if l else f0
                        w1 = xi - f.astype(jnp.float32)
                        i1 = f + 1
                        c0 = jnp.clip(f, 0, wl - 1)
                        c1 = jnp.clip(i1, 0, wl - 1)
                        if l == 0:
                            r0v = lane + lax.shift_right_logical(c0 & 128, 3)
                            r1v = lane + lax.shift_right_logical(c1 & 128, 3)
                            v0 = plsc.load_gather(bufs[p][0], [r0v, c0 & 127])
                            v1 = plsc.load_gather(bufs[p][0], [r1v, c1 & 127])
                        elif l == 3:
                            v0 = plsc.load_gather(bufs[p][3], [lane, c0 + 64])
                            v1 = plsc.load_gather(bufs[p][3], [lane, c1 + 64])
                        else:
                            v0 = plsc.load_gather(bufs[p][l], [lane, c0])
                            v1 = plsc.load_gather(bufs[p][l], [lane, c1])
                        v0 = jnp.where(f == c0, v0, jnp.float32(0.0))
                        v1 = jnp.where(i1 == c1, v1, jnp.float32(0.0))
                        ch = l * _GS + g * _SAMPLES + s - half * hcw
                        o_v[ch, h_off, pl.ds(woff, _LANES)] = (
                            v0 + w1 * (v1 - v0))

        def run_pass(half, first):
            def body(k2, carry):
                for qp in range(2):
                    k = k2 * 2 + qp
                    wait_in(k, qp, half)
                    if qp == 1:
                        @pl.when(k2 < 31)
                        def _start_next():
                            start_in(k + 1, 0, half)
                    else:
                        start_in(k + 1, 1, half)
                    compute(k, qp, half)
                return carry

            if not first:
                out_copy(0).wait()
            lax.fori_loop(0, 32, body, 0)
            out_copy(half).start()

        for cp in coord_copies(sem_o):
            cp.start()
        start_in(0, 0, 0)
        for cp in coord_copies(sem_o):
            cp.wait()
        run_pass(0, True)
        start_in(0, 0, 1)
        run_pass(1, False)
        out_copy(1).wait()

    return sampler


def kernel(fmap1, fmap2, coords, sigma):
    b, c, h, w = fmap1.shape
    o0, o1, o23 = _corr_pyramid(fmap1, fmap2)
    sampler = _make_sc_sampler(b, h, w)
    return sampler(o0, o1, o23, coords, sigma)


# speedup vs baseline: 5.1340x; 1.0047x over previous
import functools
import math

import jax
import jax.numpy as jnp
from jax import lax
from jax.experimental import pallas as pl
from jax.experimental.pallas import tpu as pltpu
from jax.experimental.pallas import tpu_sc as plsc

_SAMPLES = 9
_NLVL = 4
_G = 4
_GS = _G * _SAMPLES
_CH = _NLVL * _GS
_LANES = 16
_HB = 8


def _pool_matrix(wl, wn, scale):
    r = lax.broadcasted_iota(jnp.int32, (wl, wn), 0)
    c = lax.broadcasted_iota(jnp.int32, (wl, wn), 1)
    return jnp.where(r // 2 == c, jnp.float32(scale), jnp.float32(0.0))


def _corr_pyr_body(f1_ref, f2_ref, o0_ref, o1_ref, o23_ref):
    c = f1_ref.shape[1]
    scale = jnp.float32(1.0 / math.sqrt(float(c)))
    p1 = _pool_matrix(256, 128, 0.5)
    r = lax.broadcasted_iota(jnp.int32, (128, 128), 0)
    cc = lax.broadcasted_iota(jnp.int32, (128, 128), 1)
    q = jnp.where(
        (cc < 64) & (r // 2 == cc), jnp.float32(0.5),
        jnp.where((cc >= 64) & (cc < 96) & (r // 4 == cc - 64),
                  jnp.float32(0.25), jnp.float32(0.0)))
    for hh in range(_HB):
        f1 = f1_ref[0, :, hh, :]
        f2 = f2_ref[0, :, hh, :]
        corr = lax.dot_general(f1, f2, (((0,), (0,)), ((), ())),
                               preferred_element_type=jnp.float32) * scale
        o0_ref[0, hh] = jnp.concatenate([corr[:, :128], corr[:, 128:]], axis=0)
        l1 = lax.dot_general(corr, p1, (((1,), (0,)), ((), ())),
                             preferred_element_type=jnp.float32)
        o1_ref[0, hh] = l1
        o23_ref[0, hh] = lax.dot_general(l1, q, (((1,), (0,)), ((), ())),
                                         preferred_element_type=jnp.float32)


def _corr_pyramid(f1, f2):
    b, c, h, w = f1.shape
    shapes = [(b, h, 2 * w, 128), (b, h, w, 128), (b, h, w, 128)]
    return pl.pallas_call(
        _corr_pyr_body,
        grid=(b, h // _HB),
        in_specs=[pl.BlockSpec((1, c, _HB, w), lambda i, j: (i, 0, j, 0))] * 2,
        out_specs=[pl.BlockSpec((1, _HB, s[2], 128), lambda i, j: (i, j, 0, 0))
                   for s in shapes],
        out_shape=[jax.ShapeDtypeStruct(s, jnp.float32) for s in shapes],
    )(f1, f2)


def _make_sc_sampler(b_sz, h_sz, w):
    level_w = [w // (2 ** l) for l in range(_NLVL)]
    n_chunks = b_sz * h_sz * (w // _LANES)
    mesh = plsc.VectorSubcoreMesh(core_axis_name="c", subcore_axis_name="s")
    wc = w // _LANES
    hcw = _CH // 2

    @functools.partial(
        pl.kernel,
        mesh=mesh,
        compiler_params=pltpu.CompilerParams(needs_layout_passes=False),
        out_type=jax.ShapeDtypeStruct((b_sz, _CH, h_sz, w), jnp.float32),
        scratch_types=(
            [pltpu.VMEM((_LANES, 256), jnp.float32) for _ in range(2)]
            + [pltpu.VMEM((_LANES, 128), jnp.float32) for _ in range(4)]
            + [pltpu.VMEM((_G, 8, 128), jnp.float32) for _ in range(2)]
            + [pltpu.VMEM((hcw, 8, 128), jnp.float32)]
            + [pltpu.SemaphoreType.DMA for _ in range(3)]
        ),
    )
    def sampler(p0_hbm, p1_hbm, p23_hbm, crd_hbm, sig_hbm, out_hbm,
                p0a, p0b, p1a, p1b, p23a, p23b,
                c_v, s_v, o_v, sem_a, sem_b, sem_o):
        wid = lax.axis_index("s") * 2 + lax.axis_index("c")
        bi = lax.shift_right_logical(wid, 4)
        hb = lax.shift_right_logical(wid, 1) & 7
        wq = wid & 1
        lane = lax.broadcasted_iota(jnp.int32, (_LANES,), 0)
        bufs = [[p0a, p1a, p23a, p23a], [p0b, p1b, p23b, p23b]]
        sem_in = [sem_a, sem_b]

        def chunk_coords(k):
            hi = hb * 8 + lax.shift_right_logical(k, 3)
            s = wq * 8 + (k & 7)
            t = (bi * h_sz + hi) * wc + s
            return hi, s, t

        def in_copies(k, p, half):
            hi, s, t = chunk_coords(k)
            r0 = s * _LANES
            if half == 0:
                p0_v = bufs[p][0]
                yield pltpu.make_async_copy(
                    p0_hbm.at[bi, hi, pl.ds(r0, _LANES)],
                    p0_v.at[:, pl.ds(0, 128)], sem_in[p])
                yield pltpu.make_async_copy(
                    p0_hbm.at[bi, hi, pl.ds(w + r0, _LANES)],
                    p0_v.at[:, pl.ds(128, 128)], sem_in[p])
                yield pltpu.make_async_copy(
                    p1_hbm.at[bi, hi, pl.ds(r0, _LANES)], bufs[p][1],
                    sem_in[p])
            else:
                yield pltpu.make_async_copy(
                    p23_hbm.at[bi, hi, pl.ds(r0, _LANES)], bufs[p][2],
                    sem_in[p])

        def coord_copies(sem):
            yield pltpu.make_async_copy(
                crd_hbm.at[bi, :, pl.ds(hb * 8, 8), pl.ds(wq * 128, 128)],
                c_v, sem)
            yield pltpu.make_async_copy(
                sig_hbm.at[bi, :, pl.ds(hb * 8, 8), pl.ds(wq * 128, 128)],
                s_v, sem)

        def start_in(k, p, half):
            for cp in in_copies(k, p, half):
                cp.start()

        def wait_in(k, p, half):
            for cp in in_copies(k, p, half):
                cp.wait()

        def out_copy(half):
            return pltpu.make_async_copy(
                o_v,
                out_hbm.at[bi, pl.ds(half * hcw, hcw),
                           pl.ds(hb * 8, 8), pl.ds(wq * 128, 128)],
                sem_o)

        def compute(k, p, half):
            h_off = lax.shift_right_logical(k, 3)
            woff = (k & 7) * _LANES
            levels = (0, 1) if half == 0 else (2, 3)
            for g in range(_G):
                cg = c_v[g, h_off, pl.ds(woff, _LANES)]
                sg = s_v[g, h_off, pl.ds(woff, _LANES)]
                for s in range(_SAMPLES):
                    x = cg + jnp.float32(s - _SAMPLES // 2) * sg
                    xt = x.astype(jnp.int32)
                    f0 = jnp.where(x < xt.astype(jnp.float32), xt - 1, xt)
                    for l in levels:
                        wl = level_w[l]
                        xi = x * jnp.float32(1.0 / (2 ** l)) if l else x
                        f = lax.shift_right_arithmetic(f0, l) if l else f0
                        w1 = xi - f.astype(jnp.float32)
                        i1 = f + 1
                        c0 = jnp.clip(f, 0, wl - 1)
                        c1 = jnp.clip(i1, 0, wl - 1)
                        if l == 3:
                            v0 = plsc.load_gather(bufs[p][3], [lane, c0 + 64])
                            v1 = plsc.load_gather(bufs[p][3], [lane, c1 + 64])
                        else:
                            v0 = plsc.load_gather(bufs[p][l], [lane, c0])
                            v1 = plsc.load_gather(bufs[p][l], [lane, c1])
                        v0 = jnp.where(f == c0, v0, jnp.float32(0.0))
                        v1 = jnp.where(i1 == c1, v1, jnp.float32(0.0))
                        ch = l * _GS + g * _SAMPLES + s - half * hcw
                        o_v[ch, h_off, pl.ds(woff, _LANES)] = (
                            v0 + w1 * (v1 - v0))

        def run_pass(half, first):
            def body(k2, carry):
                for qp in range(2):
                    k = k2 * 2 + qp
                    wait_in(k, qp, half)
                    if qp == 1:
                        @pl.when(k2 < 31)
                        def _start_next():
                            start_in(k + 1, 0, half)
                    else:
                        start_in(k + 1, 1, half)
                    compute(k, qp, half)
                return carry

            if not first:
                out_copy(0).wait()
            lax.fori_loop(0, 32, body, 0)
            out_copy(half).start()

        for cp in coord_copies(sem_o):
            cp.start()
        start_in(0, 0, 0)
        for cp in coord_copies(sem_o):
            cp.wait()
        run_pass(0, True)
        start_in(0, 0, 1)
        run_pass(1, False)
        out_copy(1).wait()

    return sampler


def kernel(fmap1, fmap2, coords, sigma):
    b, c, h, w = fmap1.shape
    o0, o1, o23 = _corr_pyramid(fmap1, fmap2)
    sampler = _make_sc_sampler(b, h, w)
    return sampler(o0, o1, o23, coords, sigma)
